# Initial kernel scaffold; baseline (speedup 1.0000x reference)
import jax
import jax.numpy as jnp
from jax.experimental import pallas as pl


def kernel(boxes, scores):
    raise NotImplementedError("write your pallas kernel here")


# speedup vs baseline: 18.2758x; 18.2758x over previous
import jax
import jax.numpy as jnp
from jax import lax
from jax.experimental import pallas as pl
from jax.experimental.pallas import tpu as pltpu

_N = 20000
_NP = 20480
_C = 2048
_K = 2000
_TH = 0.7
_OUT = 512
_ICH = 256
_JCH = 2048
_NI = _NP // _ICH
_NJ = _NP // _JCH
_F32 = jnp.float32


def _body(s2_ref, vT_ref, out_ref, sjc_ref, A_ref, cT_ref, cC_ref):
    sjc_ref[:, 0:_NJ] = jnp.transpose(s2_ref[:, :])

    def cbody(c, cT):
        r0 = c // (_JCH // _ICH)
        l0 = pl.multiple_of((c % (_JCH // _ICH)) * _ICH, _ICH)
        si = s2_ref[pl.ds(r0, 1), pl.ds(l0, _ICH)]
        iidx = c * _ICH + lax.broadcasted_iota(jnp.int32, (1, _ICH), 1)
        cnt = jnp.zeros((1, _ICH), _F32)
        for jc in range(_NJ):
            sj = sjc_ref[:, jc:jc + 1]
            jidx = jc * _JCH + lax.broadcasted_iota(jnp.int32, (_JCH, 1), 0)
            win = (sj > si) | ((sj == si) & (jidx < iidx))
            cnt = cnt + jnp.sum(win.astype(_F32), axis=0, keepdims=True)
        cntc = jnp.transpose(cnt)
        ridx = lax.broadcasted_iota(jnp.int32, (1, _C), 1).astype(_F32)
        oh = (cntc == ridx).astype(_F32)
        vch = vT_ref[:, pl.ds(pl.multiple_of(c * _ICH, _ICH), _ICH)]
        return cT + jnp.dot(vch, oh, preferred_element_type=_F32,
                            precision=lax.Precision.HIGHEST)

    cT = lax.fori_loop(0, _NI, cbody, jnp.zeros((8, _C), _F32))
    cT_ref[:, :] = cT
    cC_ref[:, :] = jnp.transpose(cT)

    y0r = cT_ref[0:1, :]
    x0r = cT_ref[1:2, :]
    y1r = cT_ref[2:3, :]
    x1r = cT_ref[3:4, :]
    arear = jnp.maximum(y1r - y0r, 0.0) * jnp.maximum(x1r - x0r, 0.0)
    jrow = lax.broadcasted_iota(jnp.int32, (1, _C), 1)

    def abody(rc, carry):
        b0 = pl.multiple_of(rc * _ICH, _ICH)
        y0c = cC_ref[pl.ds(b0, _ICH), 0:1]
        x0c = cC_ref[pl.ds(b0, _ICH), 1:2]
        y1c = cC_ref[pl.ds(b0, _ICH), 2:3]
        x1c = cC_ref[pl.ds(b0, _ICH), 3:4]
        areac = jnp.maximum(y1c - y0c, 0.0) * jnp.maximum(x1c - x0c, 0.0)
        iy = jnp.maximum(0.0, jnp.minimum(y1c, y1r) - jnp.maximum(y0c, y0r))
        ix = jnp.maximum(0.0, jnp.minimum(x1c, x1r) - jnp.maximum(x0c, x0r))
        inter = iy * ix
        union = areac + arear - inter
        iou = inter / jnp.maximum(union, 1e-5)
        icol = rc * _ICH + lax.broadcasted_iota(jnp.int32, (_ICH, 1), 0)
        A_ref[pl.ds(b0, _ICH), :] = ((iou > _TH) & (icol < jrow)).astype(_F32)
        return carry

    lax.fori_loop(0, _C // _ICH, abody, 0)

    def f(x):
        s = jnp.dot(x, A_ref[:, :], preferred_element_type=_F32)
        return (s == 0.0).astype(_F32)

    hi0 = jnp.ones((1, _C), _F32)
    lo0 = f(hi0)

    def wcond(lh):
        return jnp.any(lh[0] != lh[1])

    def wbody(lh):
        lo, _ = lh
        hi2 = f(lo)
        return (f(hi2), hi2)

    keep, _ = lax.while_loop(wcond, wbody, (lo0, hi0))

    scr = cT_ref[4:5, :]
    slot = lax.broadcasted_iota(jnp.int32, (1, _C), 1)
    real = slot < _K
    msc = jnp.where(real & (keep > 0.0), scr,
                    jnp.where(real, -1.0, -2.0))
    cC_ref[:, 5:6] = jnp.transpose(msc)

    def fbody(rc, fr):
        b0 = pl.multiple_of(rc * _ICH, _ICH)
        mcc = cC_ref[pl.ds(b0, _ICH), 5:6]
        scolc = rc * _ICH + lax.broadcasted_iota(jnp.int32, (_ICH, 1), 0)
        w = (mcc > msc) | ((mcc == msc) & (scolc < slot))
        return fr + jnp.sum(w.astype(_F32), axis=0, keepdims=True)

    frank = lax.fori_loop(0, _C // _ICH, fbody, jnp.zeros((1, _C), _F32))
    frc = jnp.transpose(frank)
    ohF = (frc == lax.broadcasted_iota(jnp.int32, (1, _OUT), 1)
           .astype(_F32)).astype(_F32)
    valT2 = jnp.concatenate(
        [cT_ref[0:4, :], msc, jnp.zeros((3, _C), _F32)], axis=0)
    out_ref[:, :] = jnp.dot(valT2, ohF, preferred_element_type=_F32,
                            precision=lax.Precision.HIGHEST)


def kernel(boxes, scores):
    s = jnp.concatenate(
        [scores.astype(_F32), jnp.full((_NP - _N,), -1.0, _F32)])
    s2d = s.reshape(_NJ, _JCH)
    bT = jnp.pad(jnp.transpose(boxes.astype(_F32)),
                 ((0, 0), (0, _NP - _N)))
    valsT = jnp.concatenate(
        [bT, s[None, :], jnp.zeros((3, _NP), _F32)], axis=0)

    outT = pl.pallas_call(
        _body,
        out_shape=jax.ShapeDtypeStruct((8, _OUT), _F32),
        scratch_shapes=[
            pltpu.VMEM((_C, 16), _F32),
            pltpu.VMEM((_C, _C), _F32),
            pltpu.VMEM((8, _C), _F32),
            pltpu.VMEM((_C, 8), _F32),
        ],
        compiler_params=pltpu.CompilerParams(
            vmem_limit_bytes=100 * 1024 * 1024),
    )(s2d, valsT)
    return jnp.transpose(outT[0:5, 0:300])


# speedup vs baseline: 76.0732x; 4.1625x over previous
import jax
import jax.numpy as jnp
from jax import lax
from jax.experimental import pallas as pl
from jax.experimental.pallas import tpu as pltpu

_N = 20000
_NP = 20480
_C = 2048
_K = 2000
_TH = 0.7
_OUT = 512
_B = 256
_NR = _NP // _C
_F32 = jnp.float32
_HI = lax.Precision.HIGHEST


def _prefix_flat(x):
    jcol = lax.broadcasted_iota(jnp.int32, (_C, 1), 0)
    parts = []
    for cc in range(_C // _B):
        irow = cc * _B + lax.broadcasted_iota(jnp.int32, (1, _B), 1)
        ut = (jcol < irow).astype(_F32)
        parts.append(jnp.dot(x, ut, preferred_element_type=_F32))
    p = jnp.concatenate(parts, axis=1)
    rowtot = jnp.sum(x, axis=1, keepdims=True)
    rj = lax.broadcasted_iota(jnp.int32, (_NR, 1), 0)
    ri = lax.broadcasted_iota(jnp.int32, (1, _NR), 1)
    utr = (rj < ri).astype(_F32)
    ro = jnp.dot(jnp.transpose(rowtot), utr,
                 preferred_element_type=_F32)
    return p + jnp.transpose(ro)


def _body(s2_ref, vT_ref, out_ref, tcol_ref, A_ref, cT_ref, cC_ref):
    bits = lax.bitcast_convert_type(s2_ref[:, :], jnp.int32)
    flat = _C * lax.broadcasted_iota(jnp.int32, (_NR, _C), 0) \
        + lax.broadcasted_iota(jnp.int32, (_NR, _C), 1)
    m = flat < _N
    strict = flat < 0
    rem = jnp.full((1, 1), float(_C), _F32)
    biota = lax.broadcasted_iota(jnp.int32, (64, 1), 0)
    for sh in (24, 18, 12, 6, 0):
        bk = jnp.bitwise_and(jnp.right_shift(bits, sh), 63)
        s_cnt = jnp.zeros((64, 1), _F32)
        for r in range(_NR):
            ge = (bk[r:r + 1, :] >= biota) & m[r:r + 1, :]
            s_cnt = s_cnt + jnp.sum(ge.astype(_F32), axis=1, keepdims=True)
        bsel = jnp.sum((s_cnt >= rem).astype(_F32), keepdims=True) - 1.0
        bkf = bk.astype(_F32)
        gt = m & (bkf > bsel)
        strict = strict | gt
        rem = rem - jnp.sum(jnp.where(gt, 1.0, 0.0), keepdims=True)
        m = m & (bkf == bsel)
    tp = _prefix_flat(m.astype(_F32))
    sel = strict | (m & (tp < rem))
    sp = _prefix_flat(sel.astype(_F32))

    qm = jnp.where(sel, sp, -1.0)
    tcol_ref[:, 0:_NR] = jnp.transpose(qm)
    siota = lax.broadcasted_iota(jnp.int32, (1, _C), 1).astype(_F32)
    cIT = jnp.zeros((8, _C), _F32)
    for r in range(_NR):
        def gbody(c4, acc, r=r):
            b0 = pl.multiple_of(c4 * 512, 512)
            qc = tcol_ref[pl.ds(b0, 512), r:r + 1]
            oh = (qc == siota).astype(_F32)
            vch = vT_ref[:, pl.ds(r * _C + b0, 512)]
            return acc + jnp.dot(vch, oh, preferred_element_type=_F32,
                                 precision=_HI)
        cIT = lax.fori_loop(0, 4, gbody, cIT)
    cC_ref[:, :] = jnp.transpose(cIT)

    srow = cIT[4:5, :]
    qrow = lax.broadcasted_iota(jnp.int32, (1, _C), 1)
    rank2 = jnp.zeros((1, _C), _F32)
    for rc in range(_C // _B):
        b0 = rc * _B
        scol = cC_ref[b0:b0 + _B, 4:5]
        pcol = b0 + lax.broadcasted_iota(jnp.int32, (_B, 1), 0)
        win = (scol > srow) | ((scol == srow) & (pcol < qrow))
        rank2 = rank2 + jnp.sum(win.astype(_F32), axis=0, keepdims=True)
    tcol_ref[:, 0:1] = jnp.transpose(rank2)
    cT = jnp.zeros((8, _C), _F32)
    for rc in range(_C // _B):
        b0 = rc * _B
        r2c = tcol_ref[b0:b0 + _B, 0:1]
        oh2 = (r2c == siota).astype(_F32)
        cT = cT + jnp.dot(cIT[:, b0:b0 + _B], oh2,
                          preferred_element_type=_F32, precision=_HI)
    cT_ref[:, :] = cT
    cC_ref[:, :] = jnp.transpose(cT)

    y0r = cT_ref[0:1, :]
    x0r = cT_ref[1:2, :]
    y1r = cT_ref[2:3, :]
    x1r = cT_ref[3:4, :]
    arear = jnp.maximum(y1r - y0r, 0.0) * jnp.maximum(x1r - x0r, 0.0)
    jrow = lax.broadcasted_iota(jnp.int32, (1, _C), 1)

    def abody(rc, carry):
        b0 = pl.multiple_of(rc * _B, _B)
        y0c = cC_ref[pl.ds(b0, _B), 0:1]
        x0c = cC_ref[pl.ds(b0, _B), 1:2]
        y1c = cC_ref[pl.ds(b0, _B), 2:3]
        x1c = cC_ref[pl.ds(b0, _B), 3:4]
        areac = jnp.maximum(y1c - y0c, 0.0) * jnp.maximum(x1c - x0c, 0.0)
        iy = jnp.maximum(0.0, jnp.minimum(y1c, y1r) - jnp.maximum(y0c, y0r))
        ix = jnp.maximum(0.0, jnp.minimum(x1c, x1r) - jnp.maximum(x0c, x0r))
        inter = iy * ix
        union = areac + arear - inter
        iou = inter / jnp.maximum(union, 1e-5)
        icol = rc * _B + lax.broadcasted_iota(jnp.int32, (_B, 1), 0)
        A_ref[pl.ds(b0, _B), :] = ((iou > _TH) & (icol < jrow)).astype(_F32)
        return carry

    lax.fori_loop(0, _C // _B, abody, 0)

    sup = jnp.zeros((1, _C), _F32)
    parts = []
    for k in range(_C // _B):
        b0 = k * _B
        akk = A_ref[b0:b0 + _B, b0:b0 + _B]
        pre = (sup[:, b0:b0 + _B] == 0.0).astype(_F32)

        def fk(x, pre=pre, akk=akk):
            s = jnp.dot(x, akk, preferred_element_type=_F32)
            return pre * (s == 0.0).astype(_F32)

        hi0 = pre
        lo0 = fk(hi0)

        def wcond(lh):
            return jnp.any(lh[0] != lh[1])

        def wbody(lh, fk=fk):
            lo, _ = lh
            hi2 = fk(lo)
            return (fk(hi2), hi2)

        keepk, _ = lax.while_loop(wcond, wbody, (lo0, hi0))
        parts.append(keepk)
        sup = sup + jnp.dot(keepk, A_ref[b0:b0 + _B, :],
                            preferred_element_type=_F32)
    keep = jnp.concatenate(parts, axis=1)

    scr = cT_ref[4:5, :]
    slot = lax.broadcasted_iota(jnp.int32, (1, _C), 1)
    real = slot < _K
    msc = jnp.where(real & (keep > 0.0), scr,
                    jnp.where(real, -1.0, -2.0))
    cC_ref[:, 5:6] = jnp.transpose(msc)

    def fbody(rc, fr):
        b0 = pl.multiple_of(rc * _B, _B)
        mcc = cC_ref[pl.ds(b0, _B), 5:6]
        scolc = rc * _B + lax.broadcasted_iota(jnp.int32, (_B, 1), 0)
        w = (mcc > msc) | ((mcc == msc) & (scolc < slot))
        return fr + jnp.sum(w.astype(_F32), axis=0, keepdims=True)

    frank = lax.fori_loop(0, _C // _B, fbody, jnp.zeros((1, _C), _F32))
    frc = jnp.transpose(frank)
    ohf = (frc == lax.broadcasted_iota(jnp.int32, (1, _OUT), 1)
           .astype(_F32)).astype(_F32)
    valt2 = jnp.concatenate(
        [cT_ref[0:4, :], msc, jnp.zeros((3, _C), _F32)], axis=0)
    out_ref[:, :] = jnp.dot(valt2, ohf, preferred_element_type=_F32,
                            precision=_HI)


def kernel(boxes, scores):
    s = jnp.concatenate(
        [scores.astype(_F32), jnp.full((_NP - _N,), -1.0, _F32)])
    s2d = s.reshape(_NR, _C)
    bT = jnp.pad(jnp.transpose(boxes.astype(_F32)),
                 ((0, 0), (0, _NP - _N)))
    valsT = jnp.concatenate(
        [bT, s[None, :], jnp.zeros((3, _NP), _F32)], axis=0)

    outT = pl.pallas_call(
        _body,
        out_shape=jax.ShapeDtypeStruct((8, _OUT), _F32),
        scratch_shapes=[
            pltpu.VMEM((_C, 16), _F32),
            pltpu.VMEM((_C, _C), _F32),
            pltpu.VMEM((8, _C), _F32),
            pltpu.VMEM((_C, 8), _F32),
        ],
        compiler_params=pltpu.CompilerParams(
            vmem_limit_bytes=100 * 1024 * 1024),
    )(s2d, valsT)
    return jnp.transpose(outT[0:5, 0:300])


# speedup vs baseline: 131.1804x; 1.7244x over previous
import jax
import jax.numpy as jnp
from jax import lax
from jax.experimental import pallas as pl
from jax.experimental.pallas import tpu as pltpu

_N = 20000
_NP = 20480
_C = 2048
_K = 2000
_TH = 0.7
_OUT = 512
_B = 256
_NR = _NP // _C
_F32 = jnp.float32
_BF16 = jnp.bfloat16
_I16 = jnp.int16


def _dot3(x, oh):
    hi = x.astype(_BF16)
    r1 = x - hi.astype(_F32)
    mid = r1.astype(_BF16)
    lo = (r1 - mid.astype(_F32)).astype(_BF16)
    acc = jnp.dot(hi, oh, preferred_element_type=_F32)
    acc = acc + jnp.dot(mid, oh, preferred_element_type=_F32)
    return acc + jnp.dot(lo, oh, preferred_element_type=_F32)


def _prefix_flat(x):
    jcol = lax.broadcasted_iota(jnp.int32, (_C, 1), 0)
    parts = []
    for cc in range(_C // _B):
        irow = cc * _B + lax.broadcasted_iota(jnp.int32, (1, _B), 1)
        ut = (jcol < irow).astype(_F32)
        parts.append(jnp.dot(x, ut, preferred_element_type=_F32))
    p = jnp.concatenate(parts, axis=1)
    rowtot = jnp.sum(x, axis=1, keepdims=True)
    rj = lax.broadcasted_iota(jnp.int32, (_NR, 1), 0)
    ri = lax.broadcasted_iota(jnp.int32, (1, _NR), 1)
    utr = (rj < ri).astype(_F32)
    ro = jnp.dot(jnp.transpose(rowtot), utr,
                 preferred_element_type=_F32)
    return p + jnp.transpose(ro)


def _body(s2_ref, vT_ref, out_ref, tcol_ref, A_ref, cT_ref, cC_ref):
    bits = lax.bitcast_convert_type(s2_ref[:, :], jnp.int32)
    flat = _C * lax.broadcasted_iota(jnp.int32, (_NR, _C), 0) \
        + lax.broadcasted_iota(jnp.int32, (_NR, _C), 1)
    m = flat < _N
    strict = flat < 0
    rem = jnp.full((1, 1), float(_C), _F32)
    biota = lax.broadcasted_iota(jnp.int32, (64, 1), 0)
    for sh in (24, 18, 12, 6, 0):
        bk = jnp.bitwise_and(jnp.right_shift(bits, sh), 63)
        s_cnt = jnp.zeros((64, 1), _F32)
        for r in range(_NR):
            ge = (bk[r:r + 1, :] >= biota) & m[r:r + 1, :]
            s_cnt = s_cnt + jnp.sum(ge.astype(_F32), axis=1, keepdims=True)
        bsel = jnp.sum((s_cnt >= rem).astype(_F32), keepdims=True) - 1.0
        bkf = bk.astype(_F32)
        gt = m & (bkf > bsel)
        strict = strict | gt
        rem = rem - jnp.sum(jnp.where(gt, 1.0, 0.0), keepdims=True)
        m = m & (bkf == bsel)
    tp = _prefix_flat(m.astype(_F32))
    sel = strict | (m & (tp < rem))
    sp = _prefix_flat(sel.astype(_F32))

    qm = jnp.where(sel, sp, -1.0)
    tcol_ref[:, 0:_NR] = jnp.transpose(qm)
    siota16 = lax.broadcasted_iota(jnp.int32, (1, _C), 1).astype(_I16)
    one_b = jnp.ones((), _BF16)
    zero_b = jnp.zeros((), _BF16)
    cIT = jnp.zeros((8, _C), _F32)
    for r in range(_NR):
        for c4 in range(4):
            b0 = c4 * 512
            qc = tcol_ref[b0:b0 + 512, r:r + 1].astype(_I16)
            oh = jnp.where(qc == siota16, one_b, zero_b)
            vch = vT_ref[:, r * _C + b0:r * _C + b0 + 512]
            cIT = cIT + _dot3(vch, oh)
    cC_ref[:, :] = jnp.transpose(cIT)

    srow = cIT[4:5, :]
    qrow = lax.broadcasted_iota(jnp.int32, (1, _C), 1)
    rank2 = jnp.zeros((1, _C), _F32)
    for rc in range(_C // _B):
        b0 = rc * _B
        scol = cC_ref[b0:b0 + _B, 4:5]
        pcol = b0 + lax.broadcasted_iota(jnp.int32, (_B, 1), 0)
        win = (scol > srow) | ((scol == srow) & (pcol < qrow))
        rank2 = rank2 + jnp.sum(win.astype(_F32), axis=0, keepdims=True)
    tcol_ref[:, 0:1] = jnp.transpose(rank2)
    cT = jnp.zeros((8, _C), _F32)
    for rc in range(_C // _B):
        b0 = rc * _B
        r2c = tcol_ref[b0:b0 + _B, 0:1].astype(_I16)
        oh2 = jnp.where(r2c == siota16, one_b, zero_b)
        cT = cT + _dot3(cIT[:, b0:b0 + _B], oh2)
    cT_ref[:, :] = cT
    cC_ref[:, :] = jnp.transpose(cT)

    y0r = cT_ref[0:1, :]
    x0r = cT_ref[1:2, :]
    y1r = cT_ref[2:3, :]
    x1r = cT_ref[3:4, :]
    arear = jnp.maximum(y1r - y0r, 0.0) * jnp.maximum(x1r - x0r, 0.0)
    jrow = lax.broadcasted_iota(jnp.int32, (1, _C), 1)

    for rc in range(_C // _B):
        b0 = rc * _B
        y0c = cC_ref[b0:b0 + _B, 0:1]
        x0c = cC_ref[b0:b0 + _B, 1:2]
        y1c = cC_ref[b0:b0 + _B, 2:3]
        x1c = cC_ref[b0:b0 + _B, 3:4]
        areac = jnp.maximum(y1c - y0c, 0.0) * jnp.maximum(x1c - x0c, 0.0)
        iy = jnp.maximum(0.0, jnp.minimum(y1c, y1r[:, b0:])
                         - jnp.maximum(y0c, y0r[:, b0:]))
        ix = jnp.maximum(0.0, jnp.minimum(x1c, x1r[:, b0:])
                         - jnp.maximum(x0c, x0r[:, b0:]))
        inter = iy * ix
        union = areac + arear[:, b0:] - inter
        iou = inter / jnp.maximum(union, 1e-5)
        icol = rc * _B + lax.broadcasted_iota(jnp.int32, (_B, 1), 0)
        A_ref[b0:b0 + _B, b0:] = \
            ((iou > _TH) & (icol < jrow[:, b0:])).astype(_F32)

    sup = jnp.zeros((1, _C), _F32)
    parts = []
    for k in range(_C // _B):
        b0 = k * _B
        akk = A_ref[b0:b0 + _B, b0:b0 + _B]
        pre = (sup[:, b0:b0 + _B] == 0.0).astype(_F32)

        def fk(x, pre=pre, akk=akk):
            s = jnp.dot(x, akk, preferred_element_type=_F32)
            return pre * (s == 0.0).astype(_F32)

        hi0 = pre
        lo0 = fk(hi0)

        def wcond(lh):
            return jnp.any(lh[0] != lh[1])

        def wbody(lh, fk=fk):
            lo, _ = lh
            hi2 = fk(lo)
            return (fk(hi2), hi2)

        keepk, _ = lax.while_loop(wcond, wbody, (lo0, hi0))
        parts.append(keepk)
        if k + 1 < _C // _B:
            tail = jnp.dot(keepk, A_ref[b0:b0 + _B, b0 + _B:],
                           preferred_element_type=_F32)
            sup = sup + jnp.concatenate(
                [jnp.zeros((1, b0 + _B), _F32), tail], axis=1)
    keep = jnp.concatenate(parts, axis=1)

    scr = cT_ref[4:5, :]
    slot = lax.broadcasted_iota(jnp.int32, (1, _C), 1)
    real = slot < _K
    msc = jnp.where(real & (keep > 0.0), scr,
                    jnp.where(real, -1.0, -2.0))
    cC_ref[:, 5:6] = jnp.transpose(msc)

    def fbody(rc, fr):
        b0 = pl.multiple_of(rc * _B, _B)
        mcc = cC_ref[pl.ds(b0, _B), 5:6]
        scolc = rc * _B + lax.broadcasted_iota(jnp.int32, (_B, 1), 0)
        w = (mcc > msc) | ((mcc == msc) & (scolc < slot))
        return fr + jnp.sum(w.astype(_F32), axis=0, keepdims=True)

    frank = lax.fori_loop(0, _C // _B, fbody, jnp.zeros((1, _C), _F32))
    frc = jnp.transpose(frank).astype(_I16)
    oiota16 = lax.broadcasted_iota(jnp.int32, (1, _OUT), 1).astype(_I16)
    ohf = jnp.where(frc == oiota16, one_b, zero_b)
    valt2 = jnp.concatenate(
        [cT_ref[0:4, :], msc, jnp.zeros((3, _C), _F32)], axis=0)
    out_ref[:, :] = _dot3(valt2, ohf)


def kernel(boxes, scores):
    s = jnp.concatenate(
        [scores.astype(_F32), jnp.full((_NP - _N,), -1.0, _F32)])
    s2d = s.reshape(_NR, _C)
    bT = jnp.pad(jnp.transpose(boxes.astype(_F32)),
                 ((0, 0), (0, _NP - _N)))
    valsT = jnp.concatenate(
        [bT, s[None, :], jnp.zeros((3, _NP), _F32)], axis=0)

    outT = pl.pallas_call(
        _body,
        out_shape=jax.ShapeDtypeStruct((8, _OUT), _F32),
        scratch_shapes=[
            pltpu.VMEM((_C, 16), _F32),
            pltpu.VMEM((_C, _C), _F32),
            pltpu.VMEM((8, _C), _F32),
            pltpu.VMEM((_C, 8), _F32),
        ],
        compiler_params=pltpu.CompilerParams(
            vmem_limit_bytes=100 * 1024 * 1024),
    )(s2d, valsT)
    return jnp.transpose(outT[0:5, 0:300])


# speedup vs baseline: 212.0263x; 1.6163x over previous
import jax
import jax.numpy as jnp
from jax import lax
from jax.experimental import pallas as pl
from jax.experimental.pallas import tpu as pltpu

_N = 20000
_NP = 20480
_C = 2048
_K = 2000
_TH = 0.7
_OUT = 512
_B = 256
_NR = _NP // _C
_F32 = jnp.float32
_BF16 = jnp.bfloat16
_I16 = jnp.int16


def _dot3(x, oh):
    hi = x.astype(_BF16)
    r1 = x - hi.astype(_F32)
    mid = r1.astype(_BF16)
    lo = (r1 - mid.astype(_F32)).astype(_BF16)
    xs = jnp.concatenate([hi, mid, lo], axis=0)
    d = jnp.dot(xs, oh, preferred_element_type=_F32)
    mr = x.shape[0]
    return d[0:mr] + d[mr:2 * mr] + d[2 * mr:3 * mr]


def _prefix_flat2(x):
    n2 = 2 * _NR
    jcol = lax.broadcasted_iota(jnp.int32, (_C, 1), 0)
    parts = []
    for cc in range(_C // _B):
        irow = cc * _B + lax.broadcasted_iota(jnp.int32, (1, _B), 1)
        ut = (jcol < irow).astype(_F32)
        parts.append(jnp.dot(x, ut, preferred_element_type=_F32))
    p = jnp.concatenate(parts, axis=1)
    rowtot = jnp.sum(x, axis=1, keepdims=True)
    rj = lax.broadcasted_iota(jnp.int32, (n2, 1), 0)
    ri = lax.broadcasted_iota(jnp.int32, (1, n2), 1)
    utr = ((rj < ri) & ((rj >= _NR) == (ri >= _NR))).astype(_F32)
    ro = jnp.dot(jnp.transpose(rowtot), utr,
                 preferred_element_type=_F32)
    return p + jnp.transpose(ro)


def _body(s2_ref, vT_ref, out_ref, tcol_ref, A_ref, cT_ref, cC_ref):
    bits = lax.bitcast_convert_type(s2_ref[:, :], jnp.int32)
    flat = _C * lax.broadcasted_iota(jnp.int32, (_NR, _C), 0) \
        + lax.broadcasted_iota(jnp.int32, (_NR, _C), 1)
    m = flat < _N
    strict = flat < 0
    rem = jnp.full((1, 1), float(_C), _F32)
    biota = lax.broadcasted_iota(jnp.int32, (64, 1), 0)
    for sh in (24, 18, 12, 6, 0):
        bk = jnp.bitwise_and(jnp.right_shift(bits, sh), 63)
        bk = jnp.where(m, bk, -1)
        s_cnt = jnp.zeros((64, 1), _F32)
        for r in range(_NR):
            ge = bk[r:r + 1, :] >= biota
            s_cnt = s_cnt + jnp.sum(ge.astype(_F32), axis=1, keepdims=True)
        bsel = jnp.sum((s_cnt >= rem).astype(_F32), keepdims=True) - 1.0
        bkf = bk.astype(_F32)
        gt = m & (bkf > bsel)
        strict = strict | gt
        rem = rem - jnp.sum(jnp.where(gt, 1.0, 0.0), keepdims=True)
        m = m & (bkf == bsel)
    pp = _prefix_flat2(jnp.concatenate(
        [m.astype(_F32), strict.astype(_F32)], axis=0))
    tp, sps = pp[0:_NR], pp[_NR:2 * _NR]
    sel = strict | (m & (tp < rem))
    sp = sps + jnp.minimum(tp, rem)

    qm = jnp.where(sel, sp, -1.0)
    tcol_ref[:, 0:_NR] = jnp.transpose(qm)
    siota16 = lax.broadcasted_iota(jnp.int32, (1, _C), 1).astype(_I16)
    one_b = jnp.ones((), _BF16)
    zero_b = jnp.zeros((), _BF16)
    cIT = jnp.zeros((8, _C), _F32)
    for r in range(_NR):
        for c4 in range(4):
            b0 = c4 * 512
            qc = tcol_ref[b0:b0 + 512, r:r + 1].astype(_I16)
            oh = jnp.where(qc == siota16, one_b, zero_b)
            vch = vT_ref[:, r * _C + b0:r * _C + b0 + 512]
            cIT = cIT + _dot3(vch, oh)
    cC_ref[:, :] = jnp.transpose(cIT)

    srow = cIT[4:5, :]
    qrow = lax.broadcasted_iota(jnp.int32, (1, _C), 1)
    rank2 = jnp.zeros((1, _C), _F32)
    for rc in range(_C // _B):
        b0 = rc * _B
        scol = cC_ref[b0:b0 + _B, 4:5]
        pcol = b0 + lax.broadcasted_iota(jnp.int32, (_B, 1), 0)
        win = (scol > srow) | ((scol == srow) & (pcol < qrow))
        rank2 = rank2 + jnp.sum(win.astype(_F32), axis=0, keepdims=True)
    tcol_ref[:, 0:1] = jnp.transpose(rank2)
    cT = jnp.zeros((8, _C), _F32)
    for rc in range(_C // _B):
        b0 = rc * _B
        r2c = tcol_ref[b0:b0 + _B, 0:1].astype(_I16)
        oh2 = jnp.where(r2c == siota16, one_b, zero_b)
        cT = cT + _dot3(cIT[:, b0:b0 + _B], oh2)
    cT_ref[:, :] = cT
    cC_ref[:, :] = jnp.transpose(cT)

    y0r = cT_ref[0:1, :]
    x0r = cT_ref[1:2, :]
    y1r = cT_ref[2:3, :]
    x1r = cT_ref[3:4, :]
    arear = jnp.maximum(y1r - y0r, 0.0) * jnp.maximum(x1r - x0r, 0.0)
    jrow = lax.broadcasted_iota(jnp.int32, (1, _C), 1)

    for rc in range(_C // _B):
        b0 = rc * _B
        y0c = cC_ref[b0:b0 + _B, 0:1]
        x0c = cC_ref[b0:b0 + _B, 1:2]
        y1c = cC_ref[b0:b0 + _B, 2:3]
        x1c = cC_ref[b0:b0 + _B, 3:4]
        areac = jnp.maximum(y1c - y0c, 0.0) * jnp.maximum(x1c - x0c, 0.0)
        iy = jnp.maximum(0.0, jnp.minimum(y1c, y1r[:, b0:])
                         - jnp.maximum(y0c, y0r[:, b0:]))
        ix = jnp.maximum(0.0, jnp.minimum(x1c, x1r[:, b0:])
                         - jnp.maximum(x0c, x0r[:, b0:]))
        inter = iy * ix
        union = areac + arear[:, b0:] - inter
        iou = inter / jnp.maximum(union, 1e-5)
        icol = rc * _B + lax.broadcasted_iota(jnp.int32, (_B, 1), 0)
        A_ref[b0:b0 + _B, b0:] = \
            ((iou > _TH) & (icol < jrow[:, b0:])).astype(_F32)

    sup = jnp.zeros((1, _C), _F32)
    parts = []
    for k in range(_C // _B):
        b0 = k * _B
        akk = A_ref[b0:b0 + _B, b0:b0 + _B]
        pre = (sup[:, b0:b0 + _B] == 0.0).astype(_F32)

        def fk(x, pre=pre, akk=akk):
            s = jnp.dot(x, akk, preferred_element_type=_F32)
            return pre * (s == 0.0).astype(_F32)

        hi0 = pre
        lo0 = fk(hi0)

        def wcond(lh):
            return jnp.any(lh[0] != lh[1])

        def wbody(lh, fk=fk):
            lo, _ = lh
            hi2 = fk(lo)
            return (fk(hi2), hi2)

        keepk, _ = lax.while_loop(wcond, wbody, (lo0, hi0))
        parts.append(keepk)
        if k + 1 < _C // _B:
            tail = jnp.dot(keepk, A_ref[b0:b0 + _B, b0 + _B:],
                           preferred_element_type=_F32)
            sup = sup + jnp.concatenate(
                [jnp.zeros((1, b0 + _B), _F32), tail], axis=1)
    keep = jnp.concatenate(parts, axis=1)

    scr = cT_ref[4:5, :]
    slot = lax.broadcasted_iota(jnp.int32, (1, _C), 1)
    real = slot < _K
    msc = jnp.where(real & (keep > 0.0), scr,
                    jnp.where(real, -1.0, -2.0))
    cC_ref[:, 5:6] = jnp.transpose(msc)

    def fbody(rc, fr):
        b0 = pl.multiple_of(rc * _B, _B)
        mcc = cC_ref[pl.ds(b0, _B), 5:6]
        scolc = rc * _B + lax.broadcasted_iota(jnp.int32, (_B, 1), 0)
        w = (mcc > msc) | ((mcc == msc) & (scolc < slot))
        return fr + jnp.sum(w.astype(_F32), axis=0, keepdims=True)

    frank = lax.fori_loop(0, _C // _B, fbody, jnp.zeros((1, _C), _F32))
    frc = jnp.transpose(frank).astype(_I16)
    oiota16 = lax.broadcasted_iota(jnp.int32, (1, _OUT), 1).astype(_I16)
    ohf = jnp.where(frc == oiota16, one_b, zero_b)
    valt2 = jnp.concatenate(
        [cT_ref[0:4, :], msc, jnp.zeros((3, _C), _F32)], axis=0)
    out_ref[:, :] = _dot3(valt2, ohf)


def kernel(boxes, scores):
    s = jnp.concatenate(
        [scores.astype(_F32), jnp.full((_NP - _N,), -1.0, _F32)])
    s2d = s.reshape(_NR, _C)
    bT = jnp.pad(jnp.transpose(boxes.astype(_F32)),
                 ((0, 0), (0, _NP - _N)))
    valsT = jnp.concatenate(
        [bT, s[None, :], jnp.zeros((3, _NP), _F32)], axis=0)

    outT = pl.pallas_call(
        _body,
        out_shape=jax.ShapeDtypeStruct((8, _OUT), _F32),
        scratch_shapes=[
            pltpu.VMEM((_C, 16), _F32),
            pltpu.VMEM((_C, _C), _F32),
            pltpu.VMEM((8, _C), _F32),
            pltpu.VMEM((_C, 8), _F32),
        ],
        compiler_params=pltpu.CompilerParams(
            vmem_limit_bytes=100 * 1024 * 1024),
    )(s2d, valsT)
    return jnp.transpose(outT[0:5, 0:300])


# speedup vs baseline: 243.3942x; 1.1479x over previous
import jax
import jax.numpy as jnp
from jax import lax
from jax.experimental import pallas as pl
from jax.experimental.pallas import tpu as pltpu

_N = 20000
_NP = 20480
_C = 2048
_K = 2000
_TH = 0.7
_OUT = 512
_B = 256
_NR = _NP // _C
_F32 = jnp.float32
_BF16 = jnp.bfloat16
_I16 = jnp.int16


def _dot3(x, oh):
    hi = x.astype(_BF16)
    r1 = x - hi.astype(_F32)
    mid = r1.astype(_BF16)
    lo = (r1 - mid.astype(_F32)).astype(_BF16)
    xs = jnp.concatenate([hi, mid, lo], axis=0)
    d = jnp.dot(xs, oh, preferred_element_type=_F32)
    mr = x.shape[0]
    return d[0:mr] + d[mr:2 * mr] + d[2 * mr:3 * mr]


def _prefix_flat2(x):
    n2 = 2 * _NR
    jcol = lax.broadcasted_iota(jnp.int32, (_C, 1), 0)
    parts = []
    for cc in range(_C // _B):
        irow = cc * _B + lax.broadcasted_iota(jnp.int32, (1, _B), 1)
        ut = (jcol < irow).astype(_F32)
        parts.append(jnp.dot(x, ut, preferred_element_type=_F32))
    p = jnp.concatenate(parts, axis=1)
    rowtot = jnp.sum(x, axis=1, keepdims=True)
    rj = lax.broadcasted_iota(jnp.int32, (n2, 1), 0)
    ri = lax.broadcasted_iota(jnp.int32, (1, n2), 1)
    utr = ((rj < ri) & ((rj >= _NR) == (ri >= _NR))).astype(_F32)
    ro = jnp.dot(jnp.transpose(rowtot), utr,
                 preferred_element_type=_F32)
    return p + jnp.transpose(ro)


def _body(s2_ref, vT_ref, out_ref, tcol_ref, A_ref, cT_ref, cC_ref):
    bits = lax.bitcast_convert_type(s2_ref[:, :], jnp.int32)
    flat = _C * lax.broadcasted_iota(jnp.int32, (_NR, _C), 0) \
        + lax.broadcasted_iota(jnp.int32, (_NR, _C), 1)
    m = flat < _N
    strict = flat < 0
    rem = jnp.full((1, 1), float(_C), _F32)
    biota = lax.broadcasted_iota(jnp.int32, (64, 1), 0)
    for sh in (24, 18, 12, 6, 0):
        bk = jnp.bitwise_and(jnp.right_shift(bits, sh), 63)
        bk = jnp.where(m, bk, -1)
        s_cnt = jnp.zeros((64, 1), _F32)
        for r in range(_NR):
            ge = bk[r:r + 1, :] >= biota
            s_cnt = s_cnt + jnp.sum(ge.astype(_F32), axis=1, keepdims=True)
        bsel = jnp.sum((s_cnt >= rem).astype(_F32), keepdims=True) - 1.0
        bkf = bk.astype(_F32)
        gt = m & (bkf > bsel)
        strict = strict | gt
        rem = rem - jnp.sum(jnp.where(gt, 1.0, 0.0), keepdims=True)
        m = m & (bkf == bsel)
    pp = _prefix_flat2(jnp.concatenate(
        [m.astype(_F32), strict.astype(_F32)], axis=0))
    tp, sps = pp[0:_NR], pp[_NR:2 * _NR]
    sel = strict | (m & (tp < rem))
    sp = sps + jnp.minimum(tp, rem)

    qm = jnp.where(sel, sp, -1.0)
    tcol_ref[:, 0:_NR] = jnp.transpose(qm)
    siota16 = lax.broadcasted_iota(jnp.int32, (1, _C), 1).astype(_I16)
    one_b = jnp.ones((), _BF16)
    zero_b = jnp.zeros((), _BF16)
    cIT = jnp.zeros((8, _C), _F32)
    for r in range(_NR):
        for c4 in range(4):
            b0 = c4 * 512
            qc = tcol_ref[b0:b0 + 512, r:r + 1].astype(_I16)
            oh = jnp.where(qc == siota16, one_b, zero_b)
            vch = vT_ref[:, r * _C + b0:r * _C + b0 + 512]
            cIT = cIT + _dot3(vch, oh)
    cC_ref[:, :] = jnp.transpose(cIT)

    srow = cIT[4:5, :]
    qrow = lax.broadcasted_iota(jnp.int32, (1, _C), 1)
    rank2 = jnp.zeros((1, _C), _F32)
    for rc in range(_C // _B):
        b0 = rc * _B
        scol = cC_ref[b0:b0 + _B, 4:5]
        pcol = b0 + lax.broadcasted_iota(jnp.int32, (_B, 1), 0)
        win = (scol > srow) | ((scol == srow) & (pcol < qrow))
        rank2 = rank2 + jnp.sum(win.astype(_F32), axis=0, keepdims=True)
    tcol_ref[:, 0:1] = jnp.transpose(rank2)
    cT = jnp.zeros((8, _C), _F32)
    for rc in range(_C // _B):
        b0 = rc * _B
        r2c = tcol_ref[b0:b0 + _B, 0:1].astype(_I16)
        oh2 = jnp.where(r2c == siota16, one_b, zero_b)
        cT = cT + _dot3(cIT[:, b0:b0 + _B], oh2)
    cT_ref[:, :] = cT
    cC_ref[:, :] = jnp.transpose(cT)

    y0r = cT_ref[0:1, :]
    x0r = cT_ref[1:2, :]
    y1r = cT_ref[2:3, :]
    x1r = cT_ref[3:4, :]
    arear = jnp.maximum(y1r - y0r, 0.0) * jnp.maximum(x1r - x0r, 0.0)
    jrow = lax.broadcasted_iota(jnp.int32, (1, _C), 1)

    for rc in range(_C // _B):
        b0 = rc * _B
        y0c = cC_ref[b0:b0 + _B, 0:1]
        x0c = cC_ref[b0:b0 + _B, 1:2]
        y1c = cC_ref[b0:b0 + _B, 2:3]
        x1c = cC_ref[b0:b0 + _B, 3:4]
        areac = jnp.maximum(y1c - y0c, 0.0) * jnp.maximum(x1c - x0c, 0.0)
        iy = jnp.maximum(0.0, jnp.minimum(y1c, y1r[:, b0:])
                         - jnp.maximum(y0c, y0r[:, b0:]))
        ix = jnp.maximum(0.0, jnp.minimum(x1c, x1r[:, b0:])
                         - jnp.maximum(x0c, x0r[:, b0:]))
        inter = iy * ix
        union = areac + arear[:, b0:] - inter
        iou = inter / jnp.maximum(union, 1e-5)
        icol = rc * _B + lax.broadcasted_iota(jnp.int32, (_B, 1), 0)
        A_ref[b0:b0 + _B, b0:] = \
            ((iou > _TH) & (icol < jrow[:, b0:])).astype(_F32)

    sup = jnp.zeros((1, _C), _F32)
    parts = []
    for k in range(_C // _B):
        b0 = k * _B
        akk = A_ref[b0:b0 + _B, b0:b0 + _B]
        pre = (sup[:, b0:b0 + _B] == 0.0).astype(_F32)

        def fk(x, pre=pre, akk=akk):
            s = jnp.dot(x, akk, preferred_element_type=_F32)
            return pre * (s == 0.0).astype(_F32)

        hi0 = pre
        lo0 = fk(hi0)

        def wcond(lh):
            return jnp.any(lh[0] != lh[1])

        def wbody(lh, fk=fk):
            lo, _ = lh
            hi2 = fk(lo)
            return (fk(hi2), hi2)

        keepk, _ = lax.while_loop(wcond, wbody, (lo0, hi0))
        parts.append(keepk)
        if k + 1 < _C // _B:
            tail = jnp.dot(keepk, A_ref[b0:b0 + _B, b0 + _B:],
                           preferred_element_type=_F32)
            sup = sup + jnp.concatenate(
                [jnp.zeros((1, b0 + _B), _F32), tail], axis=1)
    keep = jnp.concatenate(parts, axis=1)

    scr = cT_ref[4:5, :]
    slot = lax.broadcasted_iota(jnp.int32, (1, _C), 1)
    slotf = slot.astype(_F32)
    real = slot < _K
    msc = jnp.where(real & (keep > 0.0), scr,
                    jnp.where(real, -1.0, -2.0))
    keepr = keep * jnp.where(real, 1.0, 0.0)
    jcol = lax.broadcasted_iota(jnp.int32, (_C, 1), 0)
    fparts = []
    for cc in range(_C // _B):
        irow = cc * _B + lax.broadcasted_iota(jnp.int32, (1, _B), 1)
        ut = (jcol < irow).astype(_F32)
        fparts.append(jnp.dot(keepr, ut, preferred_element_type=_F32))
    pk = jnp.concatenate(fparts, axis=1)
    nk = jnp.sum(keepr, keepdims=True)
    frank = jnp.where(keep > 0.0, pk, nk + slotf - pk)
    frank = jnp.where(real, frank, slotf)
    frc = jnp.transpose(frank).astype(_I16)
    oiota16 = lax.broadcasted_iota(jnp.int32, (1, _OUT), 1).astype(_I16)
    ohf = jnp.where(frc == oiota16, one_b, zero_b)
    valt2 = jnp.concatenate(
        [cT_ref[0:4, :], msc, jnp.zeros((3, _C), _F32)], axis=0)
    out_ref[:, :] = _dot3(valt2, ohf)


def kernel(boxes, scores):
    s = jnp.concatenate(
        [scores.astype(_F32), jnp.full((_NP - _N,), -1.0, _F32)])
    s2d = s.reshape(_NR, _C)
    bT = jnp.pad(jnp.transpose(boxes.astype(_F32)),
                 ((0, 0), (0, _NP - _N)))
    valsT = jnp.concatenate(
        [bT, s[None, :], jnp.zeros((3, _NP), _F32)], axis=0)

    outT = pl.pallas_call(
        _body,
        out_shape=jax.ShapeDtypeStruct((8, _OUT), _F32),
        scratch_shapes=[
            pltpu.VMEM((_C, 16), _F32),
            pltpu.VMEM((_C, _C), _F32),
            pltpu.VMEM((8, _C), _F32),
            pltpu.VMEM((_C, 8), _F32),
        ],
        compiler_params=pltpu.CompilerParams(
            vmem_limit_bytes=100 * 1024 * 1024),
    )(s2d, valsT)
    return jnp.transpose(outT[0:5, 0:300])


# speedup vs baseline: 246.7849x; 1.0139x over previous
import jax
import jax.numpy as jnp
from jax import lax
from jax.experimental import pallas as pl
from jax.experimental.pallas import tpu as pltpu

_N = 20000
_NP = 20480
_C = 2048
_K = 2000
_TH = 0.7
_OUT = 512
_B = 256
_NR = _NP // _C
_F32 = jnp.float32
_BF16 = jnp.bfloat16
_I16 = jnp.int16


def _dot3(x, oh):
    hi = x.astype(_BF16)
    r1 = x - hi.astype(_F32)
    mid = r1.astype(_BF16)
    lo = (r1 - mid.astype(_F32)).astype(_BF16)
    xs = jnp.concatenate([hi, mid, lo], axis=0)
    d = jnp.dot(xs, oh, preferred_element_type=_F32)
    mr = x.shape[0]
    return d[0:mr] + d[mr:2 * mr] + d[2 * mr:3 * mr]


def _prefix_flat2(x):
    n2 = 2 * _NR
    jcol = lax.broadcasted_iota(jnp.int32, (_C, 1), 0)
    parts = []
    for cc in range(_C // _B):
        irow = cc * _B + lax.broadcasted_iota(jnp.int32, (1, _B), 1)
        ut = (jcol < irow).astype(_F32)
        parts.append(jnp.dot(x, ut, preferred_element_type=_F32))
    p = jnp.concatenate(parts, axis=1)
    rowtot = jnp.sum(x, axis=1, keepdims=True)
    rj = lax.broadcasted_iota(jnp.int32, (n2, 1), 0)
    ri = lax.broadcasted_iota(jnp.int32, (1, n2), 1)
    utr = ((rj < ri) & ((rj >= _NR) == (ri >= _NR))).astype(_F32)
    ro = jnp.dot(jnp.transpose(rowtot), utr,
                 preferred_element_type=_F32)
    return p + jnp.transpose(ro)


def _body(s2_ref, vT_ref, out_ref, tcol_ref, A_ref, cT_ref, cC_ref):
    bits = lax.bitcast_convert_type(s2_ref[:, :], jnp.int32)
    flat = _C * lax.broadcasted_iota(jnp.int32, (_NR, _C), 0) \
        + lax.broadcasted_iota(jnp.int32, (_NR, _C), 1)
    m = flat < _N
    strict = flat < 0
    rem = jnp.full((1, 1), float(_C), _F32)
    biota = lax.broadcasted_iota(jnp.int32, (64, 1), 0)
    for sh in (24, 18, 12, 6, 0):
        bk = jnp.bitwise_and(jnp.right_shift(bits, sh), 63)
        bk = jnp.where(m, bk, -1)
        s_cnt = jnp.zeros((64, 1), _F32)
        for r in range(_NR):
            ge = bk[r:r + 1, :] >= biota
            s_cnt = s_cnt + jnp.sum(ge.astype(_F32), axis=1, keepdims=True)
        bsel = jnp.sum((s_cnt >= rem).astype(_F32), keepdims=True) - 1.0
        bkf = bk.astype(_F32)
        gt = m & (bkf > bsel)
        strict = strict | gt
        rem = rem - jnp.sum(jnp.where(gt, 1.0, 0.0), keepdims=True)
        m = m & (bkf == bsel)
    pp = _prefix_flat2(jnp.concatenate(
        [m.astype(_F32), strict.astype(_F32)], axis=0))
    tp, sps = pp[0:_NR], pp[_NR:2 * _NR]
    sel = strict | (m & (tp < rem))
    sp = sps + jnp.minimum(tp, rem)

    qm = jnp.where(sel, sp, -1.0)
    tcol_ref[:, 0:_NR] = jnp.transpose(qm)
    siota16 = lax.broadcasted_iota(jnp.int32, (1, _C), 1).astype(_I16)
    one_b = jnp.ones((), _BF16)
    zero_b = jnp.zeros((), _BF16)
    cIT = jnp.zeros((8, _C), _F32)
    for r in range(_NR):
        for c4 in range(4):
            b0 = c4 * 512
            qc = tcol_ref[b0:b0 + 512, r:r + 1].astype(_I16)
            oh = jnp.where(qc == siota16, one_b, zero_b)
            vch = vT_ref[:, r * _C + b0:r * _C + b0 + 512]
            cIT = cIT + _dot3(vch, oh)
    cC_ref[:, :] = jnp.transpose(cIT)

    srow = cIT[4:5, :]
    qrow = lax.broadcasted_iota(jnp.int32, (1, _C), 1)
    rank2 = jnp.zeros((1, _C), _F32)
    for rc in range(_C // _B):
        b0 = rc * _B
        scol = cC_ref[b0:b0 + _B, 4:5]
        pcol = b0 + lax.broadcasted_iota(jnp.int32, (_B, 1), 0)
        win = (scol > srow) | ((scol == srow) & (pcol < qrow))
        rank2 = rank2 + jnp.sum(win.astype(_F32), axis=0, keepdims=True)
    tcol_ref[:, 0:1] = jnp.transpose(rank2)
    cT = jnp.zeros((8, _C), _F32)
    for rc in range(_C // _B):
        b0 = rc * _B
        r2c = tcol_ref[b0:b0 + _B, 0:1].astype(_I16)
        oh2 = jnp.where(r2c == siota16, one_b, zero_b)
        cT = cT + _dot3(cIT[:, b0:b0 + _B], oh2)
    cT_ref[:, :] = cT
    cC_ref[:, :] = jnp.transpose(cT)

    y0r = cT_ref[0:1, :]
    x0r = cT_ref[1:2, :]
    y1r = cT_ref[2:3, :]
    x1r = cT_ref[3:4, :]
    arear = jnp.maximum(y1r - y0r, 0.0) * jnp.maximum(x1r - x0r, 0.0)
    jrow = lax.broadcasted_iota(jnp.int32, (1, _C), 1)

    for rc in range(_C // _B):
        b0 = rc * _B
        y0c = cC_ref[b0:b0 + _B, 0:1]
        x0c = cC_ref[b0:b0 + _B, 1:2]
        y1c = cC_ref[b0:b0 + _B, 2:3]
        x1c = cC_ref[b0:b0 + _B, 3:4]
        areac = jnp.maximum(y1c - y0c, 0.0) * jnp.maximum(x1c - x0c, 0.0)
        iy = jnp.maximum(0.0, jnp.minimum(y1c, y1r[:, b0:])
                         - jnp.maximum(y0c, y0r[:, b0:]))
        ix = jnp.maximum(0.0, jnp.minimum(x1c, x1r[:, b0:])
                         - jnp.maximum(x0c, x0r[:, b0:]))
        inter = iy * ix
        union = areac + arear[:, b0:] - inter
        iou = inter / jnp.maximum(union, 1e-5)
        icol = rc * _B + lax.broadcasted_iota(jnp.int32, (_B, 1), 0)
        A_ref[b0:b0 + _B, b0:] = \
            ((iou > _TH) & (icol < jrow[:, b0:])).astype(_F32)

    _NB = 512
    sup = jnp.zeros((1, _C), _F32)
    parts = []
    for k in range(_C // _NB):
        b0 = k * _NB
        akk = A_ref[b0:b0 + _NB, b0:b0 + _NB]
        pre = (sup[:, b0:b0 + _NB] == 0.0).astype(_F32)

        def fk(x, pre=pre, akk=akk):
            s = jnp.dot(x, akk, preferred_element_type=_F32)
            return pre * (s == 0.0).astype(_F32)

        hi0 = pre
        lo0 = fk(hi0)

        def wcond(lh):
            return jnp.any(lh[0] != lh[1])

        def wbody(lh, fk=fk):
            lo, _ = lh
            hi2 = fk(lo)
            return (fk(hi2), hi2)

        keepk, _ = lax.while_loop(wcond, wbody, (lo0, hi0))
        parts.append(keepk)
        if k + 1 < _C // _NB:
            tail = jnp.dot(keepk, A_ref[b0:b0 + _NB, b0 + _NB:],
                           preferred_element_type=_F32)
            sup = sup + jnp.concatenate(
                [jnp.zeros((1, b0 + _NB), _F32), tail], axis=1)
    keep = jnp.concatenate(parts, axis=1)

    scr = cT_ref[4:5, :]
    slot = lax.broadcasted_iota(jnp.int32, (1, _C), 1)
    slotf = slot.astype(_F32)
    real = slot < _K
    msc = jnp.where(real & (keep > 0.0), scr,
                    jnp.where(real, -1.0, -2.0))
    keepr = keep * jnp.where(real, 1.0, 0.0)
    jcol = lax.broadcasted_iota(jnp.int32, (_C, 1), 0)
    fparts = []
    for cc in range(_C // _B):
        irow = cc * _B + lax.broadcasted_iota(jnp.int32, (1, _B), 1)
        ut = (jcol < irow).astype(_F32)
        fparts.append(jnp.dot(keepr, ut, preferred_element_type=_F32))
    pk = jnp.concatenate(fparts, axis=1)
    nk = jnp.sum(keepr, keepdims=True)
    frank = jnp.where(keep > 0.0, pk, nk + slotf - pk)
    frank = jnp.where(real, frank, slotf)
    frc = jnp.transpose(frank).astype(_I16)
    oiota16 = lax.broadcasted_iota(jnp.int32, (1, _OUT), 1).astype(_I16)
    ohf = jnp.where(frc == oiota16, one_b, zero_b)
    valt2 = jnp.concatenate(
        [cT_ref[0:4, :], msc, jnp.zeros((3, _C), _F32)], axis=0)
    out_ref[:, :] = _dot3(valt2, ohf)


def kernel(boxes, scores):
    s = jnp.concatenate(
        [scores.astype(_F32), jnp.full((_NP - _N,), -1.0, _F32)])
    s2d = s.reshape(_NR, _C)
    bT = jnp.pad(jnp.transpose(boxes.astype(_F32)),
                 ((0, 0), (0, _NP - _N)))
    valsT = jnp.concatenate(
        [bT, s[None, :], jnp.zeros((3, _NP), _F32)], axis=0)

    outT = pl.pallas_call(
        _body,
        out_shape=jax.ShapeDtypeStruct((8, _OUT), _F32),
        scratch_shapes=[
            pltpu.VMEM((_C, 16), _F32),
            pltpu.VMEM((_C, _C), _F32),
            pltpu.VMEM((8, _C), _F32),
            pltpu.VMEM((_C, 8), _F32),
        ],
        compiler_params=pltpu.CompilerParams(
            vmem_limit_bytes=100 * 1024 * 1024),
    )(s2d, valsT)
    return jnp.transpose(outT[0:5, 0:300])


# speedup vs baseline: 290.6462x; 1.1777x over previous
import jax
import jax.numpy as jnp
from jax import lax
from jax.experimental import pallas as pl
from jax.experimental.pallas import tpu as pltpu

_N = 20000
_NP = 20480
_C = 2048
_K = 2000
_TH = 0.7
_OUT = 512
_B = 256
_NR = _NP // _C
_F32 = jnp.float32
_BF16 = jnp.bfloat16
_I16 = jnp.int16


def _dot3(x, oh):
    hi = x.astype(_BF16)
    r1 = x - hi.astype(_F32)
    mid = r1.astype(_BF16)
    lo = (r1 - mid.astype(_F32)).astype(_BF16)
    xs = jnp.concatenate([hi, mid, lo], axis=0)
    d = jnp.dot(xs, oh, preferred_element_type=_F32)
    mr = x.shape[0]
    return d[0:mr] + d[mr:2 * mr] + d[2 * mr:3 * mr]


def _prefix_flat2(x):
    n2 = 2 * _NR
    jcol = lax.broadcasted_iota(jnp.int32, (_C, 1), 0)
    parts = []
    for cc in range(_C // _B):
        irow = cc * _B + lax.broadcasted_iota(jnp.int32, (1, _B), 1)
        ut = (jcol < irow).astype(_F32)
        parts.append(jnp.dot(x, ut, preferred_element_type=_F32))
    p = jnp.concatenate(parts, axis=1)
    rowtot = jnp.sum(x, axis=1, keepdims=True)
    rj = lax.broadcasted_iota(jnp.int32, (n2, 1), 0)
    ri = lax.broadcasted_iota(jnp.int32, (1, n2), 1)
    utr = ((rj < ri) & ((rj >= _NR) == (ri >= _NR))).astype(_F32)
    ro = jnp.dot(jnp.transpose(rowtot), utr,
                 preferred_element_type=_F32)
    return p + jnp.transpose(ro)


def _body(s2_ref, vT_ref, out_ref, tcol_ref, A_ref, cT_ref, cC_ref):
    bits = lax.bitcast_convert_type(s2_ref[:, :], jnp.int32)
    flat = _C * lax.broadcasted_iota(jnp.int32, (_NR, _C), 0) \
        + lax.broadcasted_iota(jnp.int32, (_NR, _C), 1)
    m = flat < _N
    strict = flat < 0
    rem = jnp.full((1, 1), float(_C), _F32)
    biota = lax.broadcasted_iota(jnp.int32, (64, 1), 0)
    for sh in (24, 18, 12, 6, 0):
        bk = jnp.bitwise_and(jnp.right_shift(bits, sh), 63)
        bk = jnp.where(m, bk, -1)
        s_cnt = jnp.zeros((64, 1), _F32)
        for r in range(_NR):
            ge = bk[r:r + 1, :] >= biota
            s_cnt = s_cnt + jnp.sum(ge.astype(_F32), axis=1, keepdims=True)
        bsel = jnp.sum((s_cnt >= rem).astype(_F32), keepdims=True) - 1.0
        bkf = bk.astype(_F32)
        gt = m & (bkf > bsel)
        strict = strict | gt
        rem = rem - jnp.sum(jnp.where(gt, 1.0, 0.0), keepdims=True)
        m = m & (bkf == bsel)
    pp = _prefix_flat2(jnp.concatenate(
        [m.astype(_F32), strict.astype(_F32)], axis=0))
    tp, sps = pp[0:_NR], pp[_NR:2 * _NR]
    sel = strict | (m & (tp < rem))
    sp = sps + jnp.minimum(tp, rem)

    qm = jnp.where(sel, sp, -1.0)
    tcol_ref[:, 0:_NR] = jnp.transpose(qm)
    siota16 = lax.broadcasted_iota(jnp.int32, (1, _C), 1).astype(_I16)
    wiota16 = lax.broadcasted_iota(jnp.int32, (1, 1024), 1).astype(_I16)
    one_b = jnp.ones((), _BF16)
    zero_b = jnp.zeros((), _BF16)
    cI_ref = cT_ref
    cI_ref[:, :] = jnp.zeros((8, _C), _F32)
    for r in range(_NR):
        for c4 in range(4):
            b0 = c4 * 512
            qc = tcol_ref[b0:b0 + 512, r:r + 1]
            lov = lax.slice(sp, (r, b0), (r + 1, b0 + 1))
            w0 = jnp.minimum(
                (jnp.sum(lov).astype(jnp.int32) // 512) * 512, _C - 1024)
            w0 = pl.multiple_of(w0, 512)
            qrel = (qc - w0.astype(_F32)).astype(_I16)
            oh = jnp.where(qrel == wiota16, one_b, zero_b)
            vch = vT_ref[:, r * _C + b0:r * _C + b0 + 512]
            acc = cI_ref[:, pl.ds(w0, 1024)]
            cI_ref[:, pl.ds(w0, 1024)] = acc + _dot3(vch, oh)
    cIT = cI_ref[:, :]
    cC_ref[:, :] = jnp.transpose(cIT)

    srow = cIT[4:5, :]
    qrow = lax.broadcasted_iota(jnp.int32, (1, _C), 1)
    rank2 = jnp.zeros((1, _C), _F32)
    for rc in range(_C // _B):
        b0 = rc * _B
        scol = cC_ref[b0:b0 + _B, 4:5]
        pcol = b0 + lax.broadcasted_iota(jnp.int32, (_B, 1), 0)
        win = (scol > srow) | ((scol == srow) & (pcol < qrow))
        rank2 = rank2 + jnp.sum(win.astype(_F32), axis=0, keepdims=True)
    tcol_ref[:, 0:1] = jnp.transpose(rank2)
    cT = jnp.zeros((8, _C), _F32)
    for rc in range(_C // _B):
        b0 = rc * _B
        r2c = tcol_ref[b0:b0 + _B, 0:1].astype(_I16)
        oh2 = jnp.where(r2c == siota16, one_b, zero_b)
        cT = cT + _dot3(cIT[:, b0:b0 + _B], oh2)
    cT_ref[:, :] = cT
    cC_ref[:, :] = jnp.transpose(cT)

    y0r = cT_ref[0:1, :]
    x0r = cT_ref[1:2, :]
    y1r = cT_ref[2:3, :]
    x1r = cT_ref[3:4, :]
    arear = jnp.maximum(y1r - y0r, 0.0) * jnp.maximum(x1r - x0r, 0.0)
    jrow = lax.broadcasted_iota(jnp.int32, (1, _C), 1)

    for rc in range(_C // _B):
        b0 = rc * _B
        y0c = cC_ref[b0:b0 + _B, 0:1]
        x0c = cC_ref[b0:b0 + _B, 1:2]
        y1c = cC_ref[b0:b0 + _B, 2:3]
        x1c = cC_ref[b0:b0 + _B, 3:4]
        areac = jnp.maximum(y1c - y0c, 0.0) * jnp.maximum(x1c - x0c, 0.0)
        iy = jnp.maximum(0.0, jnp.minimum(y1c, y1r[:, b0:])
                         - jnp.maximum(y0c, y0r[:, b0:]))
        ix = jnp.maximum(0.0, jnp.minimum(x1c, x1r[:, b0:])
                         - jnp.maximum(x0c, x0r[:, b0:]))
        inter = iy * ix
        union = areac + arear[:, b0:] - inter
        iou = inter / jnp.maximum(union, 1e-5)
        icol = rc * _B + lax.broadcasted_iota(jnp.int32, (_B, 1), 0)
        A_ref[b0:b0 + _B, b0:] = \
            ((iou > _TH) & (icol < jrow[:, b0:])).astype(_F32)

    _NB = _B
    sup = jnp.zeros((1, _C), _F32)
    parts = []
    for k in range(_C // _NB):
        b0 = k * _NB
        akk = A_ref[b0:b0 + _NB, b0:b0 + _NB]
        pre = (sup[:, b0:b0 + _NB] == 0.0).astype(_F32)

        def fk(x, pre=pre, akk=akk):
            s = jnp.dot(x, akk, preferred_element_type=_F32)
            return pre * (s == 0.0).astype(_F32)

        hi0 = pre
        lo0 = fk(hi0)

        def wcond(lh):
            return jnp.any(lh[0] != lh[1])

        def wbody(lh, fk=fk):
            lo, _ = lh
            hi2 = fk(lo)
            return (fk(hi2), hi2)

        keepk, _ = lax.while_loop(wcond, wbody, (lo0, hi0))
        parts.append(keepk)
        if k + 1 < _C // _NB:
            tail = jnp.dot(keepk, A_ref[b0:b0 + _NB, b0 + _NB:],
                           preferred_element_type=_F32)
            sup = sup + jnp.concatenate(
                [jnp.zeros((1, b0 + _NB), _F32), tail], axis=1)
    keep = jnp.concatenate(parts, axis=1)

    scr = cT_ref[4:5, :]
    slot = lax.broadcasted_iota(jnp.int32, (1, _C), 1)
    slotf = slot.astype(_F32)
    real = slot < _K
    msc = jnp.where(real & (keep > 0.0), scr,
                    jnp.where(real, -1.0, -2.0))
    keepr = keep * jnp.where(real, 1.0, 0.0)
    jcol = lax.broadcasted_iota(jnp.int32, (_C, 1), 0)
    fparts = []
    for cc in range(_C // _B):
        irow = cc * _B + lax.broadcasted_iota(jnp.int32, (1, _B), 1)
        ut = (jcol < irow).astype(_F32)
        fparts.append(jnp.dot(keepr, ut, preferred_element_type=_F32))
    pk = jnp.concatenate(fparts, axis=1)
    nk = jnp.sum(keepr, keepdims=True)
    frank = jnp.where(keep > 0.0, pk, nk + slotf - pk)
    frank = jnp.where(real, frank, slotf)
    frc = jnp.transpose(frank).astype(_I16)
    oiota16 = lax.broadcasted_iota(jnp.int32, (1, _OUT), 1).astype(_I16)
    ohf = jnp.where(frc == oiota16, one_b, zero_b)
    valt2 = jnp.concatenate(
        [cT_ref[0:4, :], msc, jnp.zeros((3, _C), _F32)], axis=0)
    out_ref[:, :] = _dot3(valt2, ohf)


def kernel(boxes, scores):
    s = jnp.concatenate(
        [scores.astype(_F32), jnp.full((_NP - _N,), -1.0, _F32)])
    s2d = s.reshape(_NR, _C)
    bT = jnp.pad(jnp.transpose(boxes.astype(_F32)),
                 ((0, 0), (0, _NP - _N)))
    valsT = jnp.concatenate(
        [bT, s[None, :], jnp.zeros((3, _NP), _F32)], axis=0)

    outT = pl.pallas_call(
        _body,
        out_shape=jax.ShapeDtypeStruct((8, _OUT), _F32),
        scratch_shapes=[
            pltpu.VMEM((_C, 16), _F32),
            pltpu.VMEM((_C, _C), _F32),
            pltpu.VMEM((8, _C), _F32),
            pltpu.VMEM((_C, 8), _F32),
        ],
        compiler_params=pltpu.CompilerParams(
            vmem_limit_bytes=100 * 1024 * 1024),
    )(s2d, valsT)
    return jnp.transpose(outT[0:5, 0:300])


# speedup vs baseline: 319.7910x; 1.1003x over previous
import jax
import jax.numpy as jnp
from jax import lax
from jax.experimental import pallas as pl
from jax.experimental.pallas import tpu as pltpu

_N = 20000
_NP = 20480
_C = 2048
_K = 2000
_TH = 0.7
_OUT = 512
_B = 256
_NR = _NP // _C
_F32 = jnp.float32
_BF16 = jnp.bfloat16
_I16 = jnp.int16


def _dot3(x, oh):
    hi = x.astype(_BF16)
    r1 = x - hi.astype(_F32)
    mid = r1.astype(_BF16)
    lo = (r1 - mid.astype(_F32)).astype(_BF16)
    xs = jnp.concatenate([hi, mid, lo], axis=0)
    d = jnp.dot(xs, oh, preferred_element_type=_F32)
    mr = x.shape[0]
    return d[0:mr] + d[mr:2 * mr] + d[2 * mr:3 * mr]


def _prefix_flat2(x):
    n2 = 2 * _NR
    jcol = lax.broadcasted_iota(jnp.int32, (_C, 1), 0)
    parts = []
    for cc in range(_C // _B):
        irow = cc * _B + lax.broadcasted_iota(jnp.int32, (1, _B), 1)
        ut = (jcol < irow).astype(_F32)
        parts.append(jnp.dot(x, ut, preferred_element_type=_F32))
    p = jnp.concatenate(parts, axis=1)
    rowtot = jnp.sum(x, axis=1, keepdims=True)
    rj = lax.broadcasted_iota(jnp.int32, (n2, 1), 0)
    ri = lax.broadcasted_iota(jnp.int32, (1, n2), 1)
    utr = ((rj < ri) & ((rj >= _NR) == (ri >= _NR))).astype(_F32)
    ro = jnp.dot(jnp.transpose(rowtot), utr,
                 preferred_element_type=_F32)
    return p + jnp.transpose(ro)


def _body(s2_ref, vT_ref, out_ref, tcol_ref, A_ref, cT_ref, cC_ref):
    bits = lax.bitcast_convert_type(s2_ref[:, :], jnp.int32)
    flat = _C * lax.broadcasted_iota(jnp.int32, (_NR, _C), 0) \
        + lax.broadcasted_iota(jnp.int32, (_NR, _C), 1)
    m = flat < _N
    strict = flat < 0
    rem = jnp.full((1, 1), float(_C), _F32)
    biota = lax.broadcasted_iota(jnp.int32, (64, 1), 0)
    for sh in (24, 18, 12, 6, 0):
        bk = jnp.bitwise_and(jnp.right_shift(bits, sh), 63)
        bk = jnp.where(m, bk, -1)
        s_cnt = jnp.zeros((64, 1), _F32)
        for r in range(_NR):
            ge = bk[r:r + 1, :] >= biota
            s_cnt = s_cnt + jnp.sum(ge.astype(_F32), axis=1, keepdims=True)
        bsel = jnp.sum((s_cnt >= rem).astype(_F32), keepdims=True) - 1.0
        bkf = bk.astype(_F32)
        gt = m & (bkf > bsel)
        strict = strict | gt
        rem = rem - jnp.sum(jnp.where(gt, 1.0, 0.0), keepdims=True)
        m = m & (bkf == bsel)
    pp = _prefix_flat2(jnp.concatenate(
        [m.astype(_F32), strict.astype(_F32)], axis=0))
    tp, sps = pp[0:_NR], pp[_NR:2 * _NR]
    sel = strict | (m & (tp < rem))
    sp = sps + jnp.minimum(tp, rem)

    qm = jnp.where(sel, sp, -1.0)
    tcol_ref[:, 0:_NR] = jnp.transpose(qm)
    siota16 = lax.broadcasted_iota(jnp.int32, (1, _C), 1).astype(_I16)
    wiota16 = lax.broadcasted_iota(jnp.int32, (1, 640), 1).astype(_I16)
    one_b = jnp.ones((), _BF16)
    zero_b = jnp.zeros((), _BF16)
    cI_ref = cT_ref
    cI_ref[:, :] = jnp.zeros((8, _C), _F32)
    for r in range(_NR):
        for c4 in range(4):
            b0 = c4 * 512
            qc = tcol_ref[b0:b0 + 512, r:r + 1]
            lov = lax.slice(sp, (r, b0), (r + 1, b0 + 1))
            w0 = jnp.minimum(
                (jnp.sum(lov).astype(jnp.int32) // 128) * 128, _C - 640)
            w0 = pl.multiple_of(w0, 128)
            qrel = (qc - w0.astype(_F32)).astype(_I16)
            oh = jnp.where(qrel == wiota16, one_b, zero_b)
            vch = vT_ref[:, r * _C + b0:r * _C + b0 + 512]
            acc = cI_ref[:, pl.ds(w0, 640)]
            cI_ref[:, pl.ds(w0, 640)] = acc + _dot3(vch, oh)
    cIT = cI_ref[:, :]
    cC_ref[:, :] = jnp.transpose(cIT)

    srow = cIT[4:5, :]
    qrow = lax.broadcasted_iota(jnp.int32, (1, _C), 1)
    rank2 = jnp.zeros((1, _C), _F32)
    for rc in range(_C // _B):
        b0 = rc * _B
        scol = cC_ref[b0:b0 + _B, 4:5]
        pcol = b0 + lax.broadcasted_iota(jnp.int32, (_B, 1), 0)
        win = (scol > srow) | ((scol == srow) & (pcol < qrow))
        rank2 = rank2 + jnp.sum(win.astype(_F32), axis=0, keepdims=True)
    tcol_ref[:, 0:1] = jnp.transpose(rank2)
    cT = jnp.zeros((8, _C), _F32)
    for rc in range(_C // _B):
        b0 = rc * _B
        r2c = tcol_ref[b0:b0 + _B, 0:1].astype(_I16)
        oh2 = jnp.where(r2c == siota16, one_b, zero_b)
        cT = cT + _dot3(cIT[:, b0:b0 + _B], oh2)
    cT_ref[:, :] = cT
    cC_ref[:, :] = jnp.transpose(cT)

    y0r = cT_ref[0:1, :]
    x0r = cT_ref[1:2, :]
    y1r = cT_ref[2:3, :]
    x1r = cT_ref[3:4, :]
    arear = jnp.maximum(y1r - y0r, 0.0) * jnp.maximum(x1r - x0r, 0.0)
    jrow = lax.broadcasted_iota(jnp.int32, (1, _C), 1)

    for rc in range(_C // _B):
        b0 = rc * _B
        y0c = cC_ref[b0:b0 + _B, 0:1]
        x0c = cC_ref[b0:b0 + _B, 1:2]
        y1c = cC_ref[b0:b0 + _B, 2:3]
        x1c = cC_ref[b0:b0 + _B, 3:4]
        areac = jnp.maximum(y1c - y0c, 0.0) * jnp.maximum(x1c - x0c, 0.0)
        iy = jnp.maximum(0.0, jnp.minimum(y1c, y1r[:, b0:])
                         - jnp.maximum(y0c, y0r[:, b0:]))
        ix = jnp.maximum(0.0, jnp.minimum(x1c, x1r[:, b0:])
                         - jnp.maximum(x0c, x0r[:, b0:]))
        inter = iy * ix
        union = areac + arear[:, b0:] - inter
        iou = inter / jnp.maximum(union, 1e-5)
        icol = rc * _B + lax.broadcasted_iota(jnp.int32, (_B, 1), 0)
        A_ref[b0:b0 + _B, b0:] = \
            ((iou > _TH) & (icol < jrow[:, b0:])).astype(_F32)

    _NB = _B
    sup = jnp.zeros((1, _C), _F32)
    parts = []
    for k in range(_C // _NB):
        b0 = k * _NB
        akk = A_ref[b0:b0 + _NB, b0:b0 + _NB]
        pre = (sup[:, b0:b0 + _NB] == 0.0).astype(_F32)

        def fk(x, pre=pre, akk=akk):
            s = jnp.dot(x, akk, preferred_element_type=_F32)
            return pre * (s == 0.0).astype(_F32)

        hi0 = pre
        lo0 = fk(hi0)

        def wcond(lh):
            return jnp.any(lh[0] != lh[1])

        def wbody(lh, fk=fk):
            lo, _ = lh
            hi2 = fk(lo)
            return (fk(hi2), hi2)

        keepk, _ = lax.while_loop(wcond, wbody, (lo0, hi0))
        parts.append(keepk)
        if k + 1 < _C // _NB:
            tail = jnp.dot(keepk, A_ref[b0:b0 + _NB, b0 + _NB:],
                           preferred_element_type=_F32)
            sup = sup + jnp.concatenate(
                [jnp.zeros((1, b0 + _NB), _F32), tail], axis=1)
    keep = jnp.concatenate(parts, axis=1)

    scr = cT_ref[4:5, :]
    slot = lax.broadcasted_iota(jnp.int32, (1, _C), 1)
    slotf = slot.astype(_F32)
    real = slot < _K
    msc = jnp.where(real & (keep > 0.0), scr,
                    jnp.where(real, -1.0, -2.0))
    keepr = keep * jnp.where(real, 1.0, 0.0)
    jcol = lax.broadcasted_iota(jnp.int32, (_C, 1), 0)
    fparts = []
    for cc in range(_C // _B):
        irow = cc * _B + lax.broadcasted_iota(jnp.int32, (1, _B), 1)
        ut = (jcol < irow).astype(_F32)
        fparts.append(jnp.dot(keepr, ut, preferred_element_type=_F32))
    pk = jnp.concatenate(fparts, axis=1)
    nk = jnp.sum(keepr, keepdims=True)
    frank = jnp.where(keep > 0.0, pk, nk + slotf - pk)
    frank = jnp.where(real, frank, slotf)
    frc = jnp.transpose(frank).astype(_I16)
    oiota16 = lax.broadcasted_iota(jnp.int32, (1, _OUT), 1).astype(_I16)
    ohf = jnp.where(frc == oiota16, one_b, zero_b)
    valt2 = jnp.concatenate(
        [cT_ref[0:4, :], msc, jnp.zeros((3, _C), _F32)], axis=0)
    out_ref[:, :] = _dot3(valt2, ohf)


def kernel(boxes, scores):
    s = jnp.concatenate(
        [scores.astype(_F32), jnp.full((_NP - _N,), -1.0, _F32)])
    s2d = s.reshape(_NR, _C)
    bT = jnp.pad(jnp.transpose(boxes.astype(_F32)),
                 ((0, 0), (0, _NP - _N)))
    valsT = jnp.concatenate(
        [bT, s[None, :], jnp.zeros((3, _NP), _F32)], axis=0)

    outT = pl.pallas_call(
        _body,
        out_shape=jax.ShapeDtypeStruct((8, _OUT), _F32),
        scratch_shapes=[
            pltpu.VMEM((_C, 16), _F32),
            pltpu.VMEM((_C, _C), _F32),
            pltpu.VMEM((8, _C), _F32),
            pltpu.VMEM((_C, 8), _F32),
        ],
        compiler_params=pltpu.CompilerParams(
            vmem_limit_bytes=100 * 1024 * 1024),
    )(s2d, valsT)
    return jnp.transpose(outT[0:5, 0:300])


# speedup vs baseline: 324.1723x; 1.0137x over previous
import jax
import jax.numpy as jnp
from jax import lax
from jax.experimental import pallas as pl
from jax.experimental.pallas import tpu as pltpu

_N = 20000
_NP = 20480
_C = 2048
_K = 2000
_TH = 0.7
_OUT = 512
_B = 256
_NR = _NP // _C
_F32 = jnp.float32
_BF16 = jnp.bfloat16
_I16 = jnp.int16


def _dot3(x, oh):
    hi = x.astype(_BF16)
    r1 = x - hi.astype(_F32)
    mid = r1.astype(_BF16)
    lo = (r1 - mid.astype(_F32)).astype(_BF16)
    xs = jnp.concatenate([hi, mid, lo], axis=0)
    d = jnp.dot(xs, oh, preferred_element_type=_F32)
    mr = x.shape[0]
    return d[0:mr] + d[mr:2 * mr] + d[2 * mr:3 * mr]


def _prefix_flat2(x):
    n2 = 2 * _NR
    jcol = lax.broadcasted_iota(jnp.int32, (_C, 1), 0)
    parts = []
    for cc in range(_C // _B):
        irow = cc * _B + lax.broadcasted_iota(jnp.int32, (1, _B), 1)
        ut = (jcol < irow).astype(_F32)
        parts.append(jnp.dot(x, ut, preferred_element_type=_F32))
    p = jnp.concatenate(parts, axis=1)
    rowtot = jnp.sum(x, axis=1, keepdims=True)
    rj = lax.broadcasted_iota(jnp.int32, (n2, 1), 0)
    ri = lax.broadcasted_iota(jnp.int32, (1, n2), 1)
    utr = ((rj < ri) & ((rj >= _NR) == (ri >= _NR))).astype(_F32)
    ro = jnp.dot(jnp.transpose(rowtot), utr,
                 preferred_element_type=_F32)
    return p + jnp.transpose(ro)


def _body(s2_ref, vT_ref, out_ref, tcol_ref, A_ref, cT_ref, cC_ref):
    bits = lax.bitcast_convert_type(s2_ref[:, :], jnp.int32)
    flat = _C * lax.broadcasted_iota(jnp.int32, (_NR, _C), 0) \
        + lax.broadcasted_iota(jnp.int32, (_NR, _C), 1)
    m = flat < _N
    strict = flat < 0
    rem = jnp.full((1, 1), float(_C), _F32)
    biota = lax.broadcasted_iota(jnp.int32, (64, 1), 0)
    for sh in (24, 18, 12, 6, 0):
        bk = jnp.bitwise_and(jnp.right_shift(bits, sh), 63)
        bk = jnp.where(m, bk, -1)
        s_cnt = jnp.zeros((64, 1), _F32)
        for r in range(_NR):
            ge = bk[r:r + 1, :] >= biota
            s_cnt = s_cnt + jnp.sum(ge.astype(_F32), axis=1, keepdims=True)
        bsel = jnp.sum((s_cnt >= rem).astype(_F32), keepdims=True) - 1.0
        bkf = bk.astype(_F32)
        gt = m & (bkf > bsel)
        strict = strict | gt
        rem = rem - jnp.sum(jnp.where(gt, 1.0, 0.0), keepdims=True)
        m = m & (bkf == bsel)
    pp = _prefix_flat2(jnp.concatenate(
        [m.astype(_F32), strict.astype(_F32)], axis=0))
    tp, sps = pp[0:_NR], pp[_NR:2 * _NR]
    sel = strict | (m & (tp < rem))
    sp = sps + jnp.minimum(tp, rem)

    qm = jnp.where(sel, sp, -1.0)
    tcol_ref[:, 0:_NR] = jnp.transpose(qm)
    siota16 = lax.broadcasted_iota(jnp.int32, (1, _C), 1).astype(_I16)
    wiota16 = lax.broadcasted_iota(jnp.int32, (1, 384), 1).astype(_I16)
    one_b = jnp.ones((), _BF16)
    zero_b = jnp.zeros((), _BF16)
    cI_ref = cT_ref
    cI_ref[:, :] = jnp.zeros((8, _C), _F32)
    for r in range(_NR):
        for c8 in range(8):
            b0 = c8 * _B
            qc = tcol_ref[b0:b0 + _B, r:r + 1]
            lov = lax.slice(sp, (r, b0), (r + 1, b0 + 1))
            w0 = jnp.minimum(
                (jnp.sum(lov).astype(jnp.int32) // 128) * 128, _C - 384)
            w0 = pl.multiple_of(w0, 128)
            qrel = (qc - w0.astype(_F32)).astype(_I16)
            oh = jnp.where(qrel == wiota16, one_b, zero_b)
            vch = vT_ref[:, r * _C + b0:r * _C + b0 + _B]
            acc = cI_ref[:, pl.ds(w0, 384)]
            cI_ref[:, pl.ds(w0, 384)] = acc + _dot3(vch, oh)
    cIT = cI_ref[:, :]
    cC_ref[:, :] = jnp.transpose(cIT)

    srow = cIT[4:5, :]
    qrow = lax.broadcasted_iota(jnp.int32, (1, _C), 1)
    rank2 = jnp.zeros((1, _C), _F32)
    for rc in range(_C // _B):
        b0 = rc * _B
        scol = cC_ref[b0:b0 + _B, 4:5]
        pcol = b0 + lax.broadcasted_iota(jnp.int32, (_B, 1), 0)
        win = (scol > srow) | ((scol == srow) & (pcol < qrow))
        rank2 = rank2 + jnp.sum(win.astype(_F32), axis=0, keepdims=True)
    tcol_ref[:, 0:1] = jnp.transpose(rank2)
    cT = jnp.zeros((8, _C), _F32)
    for rc in range(_C // _B):
        b0 = rc * _B
        r2c = tcol_ref[b0:b0 + _B, 0:1].astype(_I16)
        oh2 = jnp.where(r2c == siota16, one_b, zero_b)
        cT = cT + _dot3(cIT[:, b0:b0 + _B], oh2)
    cT_ref[:, :] = cT
    cC_ref[:, :] = jnp.transpose(cT)

    y0r = cT_ref[0:1, :]
    x0r = cT_ref[1:2, :]
    y1r = cT_ref[2:3, :]
    x1r = cT_ref[3:4, :]
    arear = jnp.maximum(y1r - y0r, 0.0) * jnp.maximum(x1r - x0r, 0.0)
    jrow = lax.broadcasted_iota(jnp.int32, (1, _C), 1)

    for rc in range(_C // _B):
        b0 = rc * _B
        y0c = cC_ref[b0:b0 + _B, 0:1]
        x0c = cC_ref[b0:b0 + _B, 1:2]
        y1c = cC_ref[b0:b0 + _B, 2:3]
        x1c = cC_ref[b0:b0 + _B, 3:4]
        areac = jnp.maximum(y1c - y0c, 0.0) * jnp.maximum(x1c - x0c, 0.0)
        iy = jnp.maximum(0.0, jnp.minimum(y1c, y1r[:, b0:])
                         - jnp.maximum(y0c, y0r[:, b0:]))
        ix = jnp.maximum(0.0, jnp.minimum(x1c, x1r[:, b0:])
                         - jnp.maximum(x0c, x0r[:, b0:]))
        inter = iy * ix
        union = areac + arear[:, b0:] - inter
        iou = inter / jnp.maximum(union, 1e-5)
        icol = rc * _B + lax.broadcasted_iota(jnp.int32, (_B, 1), 0)
        A_ref[b0:b0 + _B, b0:] = \
            ((iou > _TH) & (icol < jrow[:, b0:])).astype(_F32)

    _NB = _B
    sup = jnp.zeros((1, _C), _F32)
    parts = []
    for k in range(_C // _NB):
        b0 = k * _NB
        akk = A_ref[b0:b0 + _NB, b0:b0 + _NB]
        pre = (sup[:, b0:b0 + _NB] == 0.0).astype(_F32)

        def fk(x, pre=pre, akk=akk):
            s = jnp.dot(x, akk, preferred_element_type=_F32)
            return pre * (s == 0.0).astype(_F32)

        hi0 = pre
        lo0 = fk(hi0)

        def wcond(lh):
            return jnp.any(lh[0] != lh[1])

        def wbody(lh, fk=fk):
            lo, _ = lh
            hi2 = fk(lo)
            return (fk(hi2), hi2)

        keepk, _ = lax.while_loop(wcond, wbody, (lo0, hi0))
        parts.append(keepk)
        if k + 1 < _C // _NB:
            tail = jnp.dot(keepk, A_ref[b0:b0 + _NB, b0 + _NB:],
                           preferred_element_type=_F32)
            sup = sup + jnp.concatenate(
                [jnp.zeros((1, b0 + _NB), _F32), tail], axis=1)
    keep = jnp.concatenate(parts, axis=1)

    scr = cT_ref[4:5, :]
    slot = lax.broadcasted_iota(jnp.int32, (1, _C), 1)
    slotf = slot.astype(_F32)
    real = slot < _K
    msc = jnp.where(real & (keep > 0.0), scr,
                    jnp.where(real, -1.0, -2.0))
    keepr = keep * jnp.where(real, 1.0, 0.0)
    jcol = lax.broadcasted_iota(jnp.int32, (_C, 1), 0)
    fparts = []
    for cc in range(_C // _B):
        irow = cc * _B + lax.broadcasted_iota(jnp.int32, (1, _B), 1)
        ut = (jcol < irow).astype(_F32)
        fparts.append(jnp.dot(keepr, ut, preferred_element_type=_F32))
    pk = jnp.concatenate(fparts, axis=1)
    nk = jnp.sum(keepr, keepdims=True)
    frank = jnp.where(keep > 0.0, pk, nk + slotf - pk)
    frank = jnp.where(real, frank, slotf)
    frc = jnp.transpose(frank).astype(_I16)
    oiota16 = lax.broadcasted_iota(jnp.int32, (1, _OUT), 1).astype(_I16)
    ohf = jnp.where(frc == oiota16, one_b, zero_b)
    valt2 = jnp.concatenate(
        [cT_ref[0:4, :], msc, jnp.zeros((3, _C), _F32)], axis=0)
    out_ref[:, :] = _dot3(valt2, ohf)


def kernel(boxes, scores):
    s = jnp.concatenate(
        [scores.astype(_F32), jnp.full((_NP - _N,), -1.0, _F32)])
    s2d = s.reshape(_NR, _C)
    bT = jnp.pad(jnp.transpose(boxes.astype(_F32)),
                 ((0, 0), (0, _NP - _N)))
    valsT = jnp.concatenate(
        [bT, s[None, :], jnp.zeros((3, _NP), _F32)], axis=0)

    outT = pl.pallas_call(
        _body,
        out_shape=jax.ShapeDtypeStruct((8, _OUT), _F32),
        scratch_shapes=[
            pltpu.VMEM((_C, 16), _F32),
            pltpu.VMEM((_C, _C), _F32),
            pltpu.VMEM((8, _C), _F32),
            pltpu.VMEM((_C, 8), _F32),
        ],
        compiler_params=pltpu.CompilerParams(
            vmem_limit_bytes=100 * 1024 * 1024),
    )(s2d, valsT)
    return jnp.transpose(outT[0:5, 0:300])


# speedup vs baseline: 326.0391x; 1.0058x over previous
import jax
import jax.numpy as jnp
from jax import lax
from jax.experimental import pallas as pl
from jax.experimental.pallas import tpu as pltpu

_N = 20000
_NP = 20480
_C = 2048
_K = 2000
_TH = 0.7
_OUT = 512
_B = 256
_NR = _NP // _C
_F32 = jnp.float32
_BF16 = jnp.bfloat16
_I16 = jnp.int16


def _dot3(x, oh):
    hi = x.astype(_BF16)
    r1 = x - hi.astype(_F32)
    mid = r1.astype(_BF16)
    lo = (r1 - mid.astype(_F32)).astype(_BF16)
    xs = jnp.concatenate([hi, mid, lo], axis=0)
    d = jnp.dot(xs, oh, preferred_element_type=_F32)
    mr = x.shape[0]
    return d[0:mr] + d[mr:2 * mr] + d[2 * mr:3 * mr]


def _prefix_flat2(x):
    n2 = 2 * _NR
    jcol = lax.broadcasted_iota(jnp.int32, (_C, 1), 0)
    parts = []
    for cc in range(_C // _B):
        irow = cc * _B + lax.broadcasted_iota(jnp.int32, (1, _B), 1)
        ut = (jcol < irow).astype(_F32)
        parts.append(jnp.dot(x, ut, preferred_element_type=_F32))
    p = jnp.concatenate(parts, axis=1)
    rowtot = jnp.sum(x, axis=1, keepdims=True)
    rj = lax.broadcasted_iota(jnp.int32, (n2, 1), 0)
    ri = lax.broadcasted_iota(jnp.int32, (1, n2), 1)
    utr = ((rj < ri) & ((rj >= _NR) == (ri >= _NR))).astype(_F32)
    ro = jnp.dot(jnp.transpose(rowtot), utr,
                 preferred_element_type=_F32)
    return p + jnp.transpose(ro)


def _body(s2_ref, vT_ref, out_ref, tcol_ref, A_ref, cT_ref, cC_ref):
    bits = lax.bitcast_convert_type(s2_ref[:, :], jnp.int32)
    flat = _C * lax.broadcasted_iota(jnp.int32, (_NR, _C), 0) \
        + lax.broadcasted_iota(jnp.int32, (_NR, _C), 1)
    m = flat < _N
    strict = flat < 0
    rem = jnp.full((1, 1), float(_C), _F32)
    biota = lax.broadcasted_iota(jnp.int32, (64, 1), 0)
    for sh in (24, 18, 12, 6, 0):
        bk = jnp.bitwise_and(jnp.right_shift(bits, sh), 63)
        bk = jnp.where(m, bk, -1)
        s_cnt = jnp.zeros((64, 1), _F32)
        for r in range(_NR):
            ge = bk[r:r + 1, :] >= biota
            s_cnt = s_cnt + jnp.sum(ge.astype(_F32), axis=1, keepdims=True)
        bsel = jnp.sum((s_cnt >= rem).astype(_F32), keepdims=True) - 1.0
        bkf = bk.astype(_F32)
        gt = m & (bkf > bsel)
        strict = strict | gt
        rem = rem - jnp.sum(jnp.where(gt, 1.0, 0.0), keepdims=True)
        m = m & (bkf == bsel)
    pp = _prefix_flat2(jnp.concatenate(
        [m.astype(_F32), strict.astype(_F32)], axis=0))
    tp, sps = pp[0:_NR], pp[_NR:2 * _NR]
    sel = strict | (m & (tp < rem))
    sp = sps + jnp.minimum(tp, rem)

    qm = jnp.where(sel, sp, -1.0)
    tcol_ref[:, 0:_NR] = jnp.transpose(qm)
    siota16 = lax.broadcasted_iota(jnp.int32, (1, _C), 1).astype(_I16)
    wiota16 = lax.broadcasted_iota(jnp.int32, (1, 384), 1).astype(_I16)
    one_b = jnp.ones((), _BF16)
    zero_b = jnp.zeros((), _BF16)
    cI_ref = cT_ref
    cI_ref[:, :] = jnp.zeros((8, _C), _F32)
    for r in range(_NR):
        for c8 in range(8):
            b0 = c8 * _B
            qc = tcol_ref[b0:b0 + _B, r:r + 1]
            lov = lax.slice(sp, (r, b0), (r + 1, b0 + 1))
            w0 = jnp.minimum(
                (jnp.sum(lov).astype(jnp.int32) // 128) * 128, _C - 384)
            w0 = pl.multiple_of(w0, 128)
            qrel = (qc - w0.astype(_F32)).astype(_I16)
            oh = jnp.where(qrel == wiota16, one_b, zero_b)
            vch = vT_ref[:, r * _C + b0:r * _C + b0 + _B]
            acc = cI_ref[:, pl.ds(w0, 384)]
            cI_ref[:, pl.ds(w0, 384)] = acc + _dot3(vch, oh)
    cIT = cI_ref[:, :]
    cC_ref[:, :] = jnp.transpose(cIT)

    srow = cIT[4:5, :]
    qrow = lax.broadcasted_iota(jnp.int32, (1, _C), 1)
    rank2 = jnp.zeros((1, _C), _F32)
    for rc in range(_C // _B):
        b0 = rc * _B
        scol = cC_ref[b0:b0 + _B, 4:5]
        pcol = b0 + lax.broadcasted_iota(jnp.int32, (_B, 1), 0)
        win = (scol > srow) | ((scol == srow) & (pcol < qrow))
        rank2 = rank2 + jnp.sum(win.astype(_F32), axis=0, keepdims=True)
    tcol_ref[:, 0:1] = jnp.transpose(rank2)
    cT = jnp.zeros((8, _C), _F32)
    for rc in range(_C // _B):
        b0 = rc * _B
        r2c = tcol_ref[b0:b0 + _B, 0:1].astype(_I16)
        oh2 = jnp.where(r2c == siota16, one_b, zero_b)
        cT = cT + _dot3(cIT[:, b0:b0 + _B], oh2)
    cT_ref[:, :] = cT
    cC_ref[:, :] = jnp.transpose(cT)

    y0r = cT_ref[0:1, :]
    x0r = cT_ref[1:2, :]
    y1r = cT_ref[2:3, :]
    x1r = cT_ref[3:4, :]
    arear = jnp.maximum(y1r - y0r, 0.0) * jnp.maximum(x1r - x0r, 0.0)
    jrow = lax.broadcasted_iota(jnp.int32, (1, _C), 1)

    for rc in range(_C // _B):
        b0 = rc * _B
        y0c = cC_ref[b0:b0 + _B, 0:1]
        x0c = cC_ref[b0:b0 + _B, 1:2]
        y1c = cC_ref[b0:b0 + _B, 2:3]
        x1c = cC_ref[b0:b0 + _B, 3:4]
        areac = jnp.maximum(y1c - y0c, 0.0) * jnp.maximum(x1c - x0c, 0.0)
        iy = jnp.maximum(0.0, jnp.minimum(y1c, y1r[:, b0:])
                         - jnp.maximum(y0c, y0r[:, b0:]))
        ix = jnp.maximum(0.0, jnp.minimum(x1c, x1r[:, b0:])
                         - jnp.maximum(x0c, x0r[:, b0:]))
        inter = iy * ix
        union = areac + arear[:, b0:] - inter
        iou = inter / jnp.maximum(union, 1e-5)
        icol = rc * _B + lax.broadcasted_iota(jnp.int32, (_B, 1), 0)
        A_ref[b0:b0 + _B, b0:] = \
            ((iou > _TH) & (icol < jrow[:, b0:])).astype(_F32)

    _NB = _B
    sup = jnp.zeros((1, _C), _F32)
    parts = []
    for k in range(_C // _NB):
        b0 = k * _NB
        akk = A_ref[b0:b0 + _NB, b0:b0 + _NB]
        pre = (sup[:, b0:b0 + _NB] == 0.0).astype(_F32)

        def fk(x, pre=pre, akk=akk):
            s = jnp.dot(x, akk, preferred_element_type=_F32)
            return pre * (s == 0.0).astype(_F32)

        hi0 = pre
        lo0 = fk(hi0)
        hi1 = fk(lo0)
        lo1 = fk(hi1)

        def wcond(lh):
            return jnp.any(lh[0] != lh[1])

        def wbody(lh, fk=fk):
            lo, _ = lh
            hi2 = fk(lo)
            return (fk(hi2), hi2)

        keepk, _ = lax.while_loop(wcond, wbody, (lo1, hi1))
        parts.append(keepk)
        if k + 1 < _C // _NB:
            tail = jnp.dot(keepk, A_ref[b0:b0 + _NB, b0 + _NB:],
                           preferred_element_type=_F32)
            sup = sup + jnp.concatenate(
                [jnp.zeros((1, b0 + _NB), _F32), tail], axis=1)
    keep = jnp.concatenate(parts, axis=1)

    scr = cT_ref[4:5, :]
    slot = lax.broadcasted_iota(jnp.int32, (1, _C), 1)
    slotf = slot.astype(_F32)
    real = slot < _K
    msc = jnp.where(real & (keep > 0.0), scr,
                    jnp.where(real, -1.0, -2.0))
    keepr = keep * jnp.where(real, 1.0, 0.0)
    jcol = lax.broadcasted_iota(jnp.int32, (_C, 1), 0)
    fparts = []
    for cc in range(_C // _B):
        irow = cc * _B + lax.broadcasted_iota(jnp.int32, (1, _B), 1)
        ut = (jcol < irow).astype(_F32)
        fparts.append(jnp.dot(keepr, ut, preferred_element_type=_F32))
    pk = jnp.concatenate(fparts, axis=1)
    nk = jnp.sum(keepr, keepdims=True)
    frank = jnp.where(keep > 0.0, pk, nk + slotf - pk)
    frank = jnp.where(real, frank, slotf)
    frc = jnp.transpose(frank).astype(_I16)
    oiota16 = lax.broadcasted_iota(jnp.int32, (1, _OUT), 1).astype(_I16)
    ohf = jnp.where(frc == oiota16, one_b, zero_b)
    valt2 = jnp.concatenate(
        [cT_ref[0:4, :], msc, jnp.zeros((3, _C), _F32)], axis=0)
    out_ref[:, :] = _dot3(valt2, ohf)


def kernel(boxes, scores):
    s = jnp.concatenate(
        [scores.astype(_F32), jnp.full((_NP - _N,), -1.0, _F32)])
    s2d = s.reshape(_NR, _C)
    bT = jnp.pad(jnp.transpose(boxes.astype(_F32)),
                 ((0, 0), (0, _NP - _N)))
    valsT = jnp.concatenate(
        [bT, s[None, :], jnp.zeros((3, _NP), _F32)], axis=0)

    outT = pl.pallas_call(
        _body,
        out_shape=jax.ShapeDtypeStruct((8, _OUT), _F32),
        scratch_shapes=[
            pltpu.VMEM((_C, 16), _F32),
            pltpu.VMEM((_C, _C), _F32),
            pltpu.VMEM((8, _C), _F32),
            pltpu.VMEM((_C, 8), _F32),
        ],
        compiler_params=pltpu.CompilerParams(
            vmem_limit_bytes=100 * 1024 * 1024),
    )(s2d, valsT)
    return jnp.transpose(outT[0:5, 0:300])
